# baseline (speedup 1.0000x reference)
import jax
import jax.numpy as jnp
from jax import lax
from jax.experimental import pallas as pl
from jax.experimental.pallas import tpu as pltpu
from jax.experimental.pallas import tpu_sc as plsc

B = 1024
S = 200
H = 128
SP = 208
NC = 2
NS = 16
NW = NC * NS
ROWS_PER_W = B // NW
LANES = 16
KCH = H // LANES
NQ = SP // LANES


def _body(ids_h, cidx_h, pos_h, typ_h, ten_h, word_h, out_h,
          pos_v, combo_v, typ_v, ids_v, cidx_v, acc_a, acc_b, g0, g1):
    c = lax.axis_index("c")
    s = lax.axis_index("s")
    wid = s * NC + c

    pltpu.sync_copy(
        ids_h.at[pl.ds(wid * (ROWS_PER_W + 1) * SP, (ROWS_PER_W + 1) * SP)],
        ids_v)
    pltpu.sync_copy(cidx_h.at[pl.ds(wid * ROWS_PER_W * SP, ROWS_PER_W * SP)],
                    cidx_v)
    pltpu.sync_copy(pos_h, pos_v)
    pltpu.sync_copy(typ_h, typ_v)
    pltpu.sync_copy(ten_h, acc_a.at[pl.ds(0, 104)])

    def build(t, carry):
        for half in range(2):
            for k in range(KCH):
                sl = pl.ds(k * LANES, LANES)
                combo_v[half * 100 + t, sl] = (
                    acc_a[t, sl] + typ_v[pl.ds(half * H + k * LANES, LANES)])
        return carry
    lax.fori_loop(0, 100, build, 0)

    def issue_gather(r, acc, sem):
        da = pltpu.async_copy(
            word_h.at[ids_v.at[pl.ds(r * SP, 104)]],
            acc.at[pl.ds(0, 104)], sem)
        db = pltpu.async_copy(
            word_h.at[ids_v.at[pl.ds(r * SP + 104, 104)]],
            acc.at[pl.ds(104, 104)], sem)
        return da, db

    def compute_wb(r, acc):
        def group(q, inner):
            t0 = q * LANES
            chunk = cidx_v[pl.ds(r * SP + t0, LANES)]
            for i in range(LANES):
                ct = chunk[i]
                t = t0 + i
                for k in range(KCH):
                    sl = pl.ds(k * LANES, LANES)
                    acc[t, sl] = acc[t, sl] + pos_v[t, sl] + combo_v[ct, sl]
            return inner
        lax.fori_loop(0, NQ, group, 0)
        with jax.named_scope("wb"):
            pltpu.sync_copy(acc.at[pl.ds(0, S)],
                            out_h.at[wid * ROWS_PER_W + r])

    da, db = issue_gather(0, acc_a, g0)
    da.wait()
    db.wait()

    def pair(p, carry):
        e = 2 * p
        with jax.named_scope("gissue"):
            d1a, d1b = issue_gather(e + 1, acc_b, g1)
        with jax.named_scope("comp"):
            compute_wb(e, acc_a)
        with jax.named_scope("gwait"):
            d1a.wait()
            d1b.wait()
        with jax.named_scope("gissue"):
            d0a, d0b = issue_gather(e + 2, acc_a, g0)
        with jax.named_scope("comp"):
            compute_wb(e + 1, acc_b)
        with jax.named_scope("gwait"):
            d0a.wait()
            d0b.wait()
        return carry
    lax.fori_loop(0, ROWS_PER_W // 2, pair, 0)


@jax.jit
def _run(ids, cidx, pos, typ, ten, word):
    mesh = plsc.VectorSubcoreMesh(core_axis_name="c", subcore_axis_name="s")
    return pl.kernel(
        _body,
        out_type=jax.ShapeDtypeStruct((B, S, H), jnp.float32),
        mesh=mesh,
        scratch_types=[
            pltpu.VMEM((SP, H), jnp.float32),
            pltpu.VMEM((S, H), jnp.float32),
            pltpu.VMEM((2 * H,), jnp.float32),
            pltpu.VMEM(((ROWS_PER_W + 1) * SP,), jnp.int32),
            pltpu.VMEM((ROWS_PER_W * SP,), jnp.int32),
            pltpu.VMEM((SP, H), jnp.float32),
            pltpu.VMEM((SP, H), jnp.float32),
            pltpu.SemaphoreType.DMA,
            pltpu.SemaphoreType.DMA,
        ],
    )(ids, cidx, pos, typ, ten, word)


def kernel(input_ids, token_type_ids, tenant_ids, W_word, W_pos, W_type, W_tenant):
    ids = input_ids.astype(jnp.int32)
    cidx = (token_type_ids.astype(jnp.int32) * 100
            + tenant_ids.astype(jnp.int32))
    ids_p = jnp.zeros((NW, ROWS_PER_W + 1, SP), jnp.int32)
    ids_p = ids_p.at[:, :ROWS_PER_W, :S].set(ids.reshape(NW, ROWS_PER_W, S))
    cidx_p = jnp.zeros((B, SP), jnp.int32).at[:, :S].set(cidx)
    pos = jnp.pad(W_pos[:S], ((0, SP - S), (0, 0)))
    ten = jnp.pad(W_tenant, ((0, 104 - W_tenant.shape[0]), (0, 0)))
    return _run(ids_p.reshape(-1), cidx_p.reshape(-1), pos,
                W_type.reshape(-1), ten, W_word)


# speedup vs baseline: 1.0028x; 1.0028x over previous
import jax
import jax.numpy as jnp
from jax import lax
from jax.experimental import pallas as pl
from jax.experimental.pallas import tpu as pltpu
from jax.experimental.pallas import tpu_sc as plsc

B = 1024
S = 200
H = 128
SP = 208
NC = 2
NS = 16
NW = NC * NS
ROWS_PER_W = B // NW
LANES = 16
KCH = H // LANES
NQ = SP // LANES


def _body(ids_h, cidx_h, pos_h, typ_h, ten_h, word_h, out_h,
          pos_v, combo_v, typ_v, cidx_v, acc_a, acc_b,
          ia0, ib0, ia1, ib1, g0, g1):
    c = lax.axis_index("c")
    s = lax.axis_index("s")
    wid = s * NC + c

    pltpu.sync_copy(cidx_h.at[pl.ds(wid * ROWS_PER_W * SP, ROWS_PER_W * SP)],
                    cidx_v)
    pltpu.sync_copy(pos_h, pos_v)
    pltpu.sync_copy(typ_h, typ_v)
    pltpu.sync_copy(ten_h, acc_a.at[pl.ds(0, 104)])

    def build(t, carry):
        for half in range(2):
            for k in range(KCH):
                sl = pl.ds(k * LANES, LANES)
                combo_v[half * 100 + t, sl] = (
                    acc_a[t, sl] + typ_v[pl.ds(half * H + k * LANES, LANES)])
        return carry
    lax.fori_loop(0, 100, build, 0)

    def issue_gather(r, acc, sem, ia, ib):
        base = (wid * (ROWS_PER_W + 1) + r) * SP
        pltpu.sync_copy(ids_h.at[pl.ds(base, 104)], ia)
        pltpu.sync_copy(ids_h.at[pl.ds(base + 104, 104)], ib)
        da = pltpu.async_copy(
            word_h.at[ia], acc.at[pl.ds(0, 104)], sem)
        db = pltpu.async_copy(
            word_h.at[ib], acc.at[pl.ds(104, 104)], sem)
        return da, db

    def compute_wb(r, acc):
        def group(q, inner):
            t0 = q * LANES
            chunk = cidx_v[pl.ds(r * SP + t0, LANES)]
            for i in range(LANES):
                ct = chunk[i]
                t = t0 + i
                for k in range(KCH):
                    sl = pl.ds(k * LANES, LANES)
                    acc[t, sl] = acc[t, sl] + pos_v[t, sl] + combo_v[ct, sl]
            return inner
        lax.fori_loop(0, NQ, group, 0)
        with jax.named_scope("wb"):
            pltpu.sync_copy(acc.at[pl.ds(0, S)],
                            out_h.at[wid * ROWS_PER_W + r])

    da, db = issue_gather(0, acc_a, g0, ia0, ib0)
    da.wait()
    db.wait()

    def pair(p, carry):
        e = 2 * p
        with jax.named_scope("gissue"):
            d1a, d1b = issue_gather(e + 1, acc_b, g1, ia1, ib1)
        with jax.named_scope("comp"):
            compute_wb(e, acc_a)
        with jax.named_scope("gwait"):
            d1a.wait()
            d1b.wait()
        with jax.named_scope("gissue"):
            d0a, d0b = issue_gather(e + 2, acc_a, g0, ia0, ib0)
        with jax.named_scope("comp"):
            compute_wb(e + 1, acc_b)
        with jax.named_scope("gwait"):
            d0a.wait()
            d0b.wait()
        return carry
    lax.fori_loop(0, ROWS_PER_W // 2, pair, 0)


@jax.jit
def _run(ids, cidx, pos, typ, ten, word):
    mesh = plsc.VectorSubcoreMesh(core_axis_name="c", subcore_axis_name="s")
    return pl.kernel(
        _body,
        out_type=jax.ShapeDtypeStruct((B, S, H), jnp.float32),
        mesh=mesh,
        scratch_types=[
            pltpu.VMEM((SP, H), jnp.float32),
            pltpu.VMEM((S, H), jnp.float32),
            pltpu.VMEM((2 * H,), jnp.float32),
            pltpu.VMEM((ROWS_PER_W * SP,), jnp.int32),
            pltpu.VMEM((SP, H), jnp.float32),
            pltpu.VMEM((SP, H), jnp.float32),
            pltpu.VMEM((104,), jnp.int32),
            pltpu.VMEM((104,), jnp.int32),
            pltpu.VMEM((104,), jnp.int32),
            pltpu.VMEM((104,), jnp.int32),
            pltpu.SemaphoreType.DMA,
            pltpu.SemaphoreType.DMA,
        ],
    )(ids, cidx, pos, typ, ten, word)


def kernel(input_ids, token_type_ids, tenant_ids, W_word, W_pos, W_type, W_tenant):
    ids = input_ids.astype(jnp.int32)
    cidx = (token_type_ids.astype(jnp.int32) * 100
            + tenant_ids.astype(jnp.int32))
    ids_p = jnp.zeros((NW, ROWS_PER_W + 1, SP), jnp.int32)
    ids_p = ids_p.at[:, :ROWS_PER_W, :S].set(ids.reshape(NW, ROWS_PER_W, S))
    cidx_p = jnp.zeros((B, SP), jnp.int32).at[:, :S].set(cidx)
    pos = jnp.pad(W_pos[:S], ((0, SP - S), (0, 0)))
    ten = jnp.pad(W_tenant, ((0, 104 - W_tenant.shape[0]), (0, 0)))
    return _run(ids_p.reshape(-1), cidx_p.reshape(-1), pos,
                W_type.reshape(-1), ten, W_word)


# speedup vs baseline: 1.4396x; 1.4356x over previous
import jax
import jax.numpy as jnp
from jax import lax
from jax.experimental import pallas as pl
from jax.experimental.pallas import tpu as pltpu
from jax.experimental.pallas import tpu_sc as plsc

B = 1024
S = 200
H = 128
SP = 208
NC = 2
NS = 16
NW = NC * NS
ROWS_PER_W = B // NW
LANES = 16
KCH = H // LANES
NQ = SP // LANES


def _body(ids_h, cidx_h, pos_h, typ_h, ten_h, word_h, out_h,
          pos_v, combo_v, typ_v, cidx_v, acc_a, acc_b,
          ia0, ib0, ia1, ib1, g0, g1):
    c = lax.axis_index("c")
    s = lax.axis_index("s")
    wid = s * NC + c

    pltpu.sync_copy(cidx_h.at[pl.ds(wid * ROWS_PER_W * SP, ROWS_PER_W * SP)],
                    cidx_v)
    pltpu.sync_copy(pos_h, pos_v)
    pltpu.sync_copy(typ_h, typ_v)
    pltpu.sync_copy(ten_h, acc_a.at[pl.ds(0, 104)])

    def build(t, carry):
        for half in range(2):
            for k in range(KCH):
                sl = pl.ds(k * LANES, LANES)
                combo_v[half * 100 + t, sl] = (
                    acc_a[t, sl] + typ_v[pl.ds(half * H + k * LANES, LANES)])
        return carry
    lax.fori_loop(0, 100, build, 0)

    def issue_gather(r, acc, sem, ia, ib):
        base = (wid * (ROWS_PER_W + 1) + r) * SP
        pltpu.sync_copy(ids_h.at[pl.ds(base, 104)], ia)
        pltpu.sync_copy(ids_h.at[pl.ds(base + 104, 104)], ib)
        da = pltpu.async_copy(
            word_h.at[ia], acc.at[pl.ds(0, 104)], sem)
        db = pltpu.async_copy(
            word_h.at[ib], acc.at[pl.ds(104, 104)], sem)
        return da, db

    def compute_wb(r, acc):
        def group(q, inner):
            t0 = q * LANES
            chunk = cidx_v[pl.ds(r * SP + t0, LANES)]
            for i in range(LANES):
                ct = chunk[i]
                t = t0 + i
                for k in range(KCH):
                    sl = pl.ds(k * LANES, LANES)
                    acc[t, sl] = acc[t, sl] + pos_v[t, sl] + combo_v[ct, sl]
            return inner
        lax.fori_loop(0, NQ, group, 0)
        with jax.named_scope("wb"):
            pltpu.sync_copy(acc.at[pl.ds(0, S)],
                            out_h.at[wid * ROWS_PER_W + r])

    def row(r, carry):
        with jax.named_scope("gath"):
            da, db = issue_gather(r, acc_a, g0, ia0, ib0)
            da.wait()
            db.wait()
        with jax.named_scope("comp"):
            compute_wb(r, acc_a)
        return carry
    lax.fori_loop(0, ROWS_PER_W, row, 0)


@jax.jit
def _run(ids, cidx, pos, typ, ten, word):
    mesh = plsc.VectorSubcoreMesh(core_axis_name="c", subcore_axis_name="s")
    return pl.kernel(
        _body,
        out_type=jax.ShapeDtypeStruct((B, S, H), jnp.float32),
        mesh=mesh,
        scratch_types=[
            pltpu.VMEM((SP, H), jnp.float32),
            pltpu.VMEM((S, H), jnp.float32),
            pltpu.VMEM((2 * H,), jnp.float32),
            pltpu.VMEM((ROWS_PER_W * SP,), jnp.int32),
            pltpu.VMEM((SP, H), jnp.float32),
            pltpu.VMEM((SP, H), jnp.float32),
            pltpu.VMEM((104,), jnp.int32),
            pltpu.VMEM((104,), jnp.int32),
            pltpu.VMEM((104,), jnp.int32),
            pltpu.VMEM((104,), jnp.int32),
            pltpu.SemaphoreType.DMA,
            pltpu.SemaphoreType.DMA,
        ],
    )(ids, cidx, pos, typ, ten, word)


def kernel(input_ids, token_type_ids, tenant_ids, W_word, W_pos, W_type, W_tenant):
    ids = input_ids.astype(jnp.int32)
    cidx = (token_type_ids.astype(jnp.int32) * 100
            + tenant_ids.astype(jnp.int32))
    ids_p = jnp.zeros((NW, ROWS_PER_W + 1, SP), jnp.int32)
    ids_p = ids_p.at[:, :ROWS_PER_W, :S].set(ids.reshape(NW, ROWS_PER_W, S))
    cidx_p = jnp.zeros((B, SP), jnp.int32).at[:, :S].set(cidx)
    pos = jnp.pad(W_pos[:S], ((0, SP - S), (0, 0)))
    ten = jnp.pad(W_tenant, ((0, 104 - W_tenant.shape[0]), (0, 0)))
    return _run(ids_p.reshape(-1), cidx_p.reshape(-1), pos,
                W_type.reshape(-1), ten, W_word)


# speedup vs baseline: 1.9550x; 1.3580x over previous
import jax
import jax.numpy as jnp
from jax import lax
from jax.experimental import pallas as pl
from jax.experimental.pallas import tpu as pltpu
from jax.experimental.pallas import tpu_sc as plsc

B = 1024
S = 200
H = 128
SPLIT_A = 104
SPLIT_B = S - SPLIT_A
NC = 2
NS = 16
NW = NC * NS
ROWS_PER_W = B // NW
LANES = 16
KCH = H // LANES
NQ = S // LANES
TAIL = S - NQ * LANES
TEN_PAD = 104


def _body(ids_h, cidx_h, pos_h, typ_h, ten_h, word_h, out_h,
          pos_v, combo_v, typ_v, ten_v, acc_v, idx_a, idx_b, cidx_v, gsem):
    c = lax.axis_index("c")
    s = lax.axis_index("s")
    wid = s * NC + c

    pltpu.sync_copy(pos_h, pos_v)
    pltpu.sync_copy(typ_h, typ_v)
    pltpu.sync_copy(ten_h, ten_v)

    def build(t, carry):
        for half in range(2):
            for k in range(KCH):
                sl = pl.ds(k * LANES, LANES)
                combo_v[half * 100 + t, sl] = (
                    ten_v[t, sl] + typ_v[pl.ds(half * H + k * LANES, LANES)])
        return carry
    lax.fori_loop(0, 100, build, 0)

    def do_token(t, ct):
        for k in range(KCH):
            sl = pl.ds(k * LANES, LANES)
            acc_v[t, sl] = acc_v[t, sl] + pos_v[t, sl] + combo_v[ct, sl]

    def row(r, carry):
        b = wid * ROWS_PER_W + r
        base = b * S
        with jax.named_scope("gid"):
            pltpu.sync_copy(ids_h.at[pl.ds(base, SPLIT_A)], idx_a)
            pltpu.sync_copy(ids_h.at[pl.ds(base + SPLIT_A, SPLIT_B)], idx_b)
            pltpu.sync_copy(cidx_h.at[pl.ds(base, S)], cidx_v.at[pl.ds(0, S)])
        with jax.named_scope("gath"):
            ga = pltpu.async_copy(word_h.at[idx_a],
                                  acc_v.at[pl.ds(0, SPLIT_A)], gsem)
            gb = pltpu.async_copy(word_h.at[idx_b],
                                  acc_v.at[pl.ds(SPLIT_A, SPLIT_B)], gsem)
            ga.wait()
            gb.wait()

        with jax.named_scope("comp"):
            def group(q, inner):
                t0 = q * LANES
                chunk = cidx_v[pl.ds(t0, LANES)]
                for i in range(LANES):
                    do_token(t0 + i, chunk[i])
                return inner
            lax.fori_loop(0, NQ, group, 0)

            tail_chunk = cidx_v[pl.ds(NQ * LANES, LANES)]
            for i in range(TAIL):
                do_token(NQ * LANES + i, tail_chunk[i])

        with jax.named_scope("wb"):
            pltpu.sync_copy(acc_v, out_h.at[b])
        return carry
    lax.fori_loop(0, ROWS_PER_W, row, 0)


@jax.jit
def _run(ids, cidx, pos, typ, ten, word):
    mesh = plsc.VectorSubcoreMesh(core_axis_name="c", subcore_axis_name="s")
    return pl.kernel(
        _body,
        out_type=jax.ShapeDtypeStruct((B, S, H), jnp.float32),
        mesh=mesh,
        scratch_types=[
            pltpu.VMEM((S, H), jnp.float32),
            pltpu.VMEM((S, H), jnp.float32),
            pltpu.VMEM((2 * H,), jnp.float32),
            pltpu.VMEM((TEN_PAD, H), jnp.float32),
            pltpu.VMEM((S, H), jnp.float32),
            pltpu.VMEM((SPLIT_A,), jnp.int32),
            pltpu.VMEM((SPLIT_B,), jnp.int32),
            pltpu.VMEM(((NQ + 1) * LANES,), jnp.int32),
            pltpu.SemaphoreType.DMA,
        ],
    )(ids, cidx, pos, typ, ten, word)


def kernel(input_ids, token_type_ids, tenant_ids, W_word, W_pos, W_type, W_tenant):
    ids = input_ids.astype(jnp.int32).reshape(B * S)
    cidx = (token_type_ids.astype(jnp.int32) * 100
            + tenant_ids.astype(jnp.int32)).reshape(B * S)
    pos = W_pos[:S]
    typ = W_type.reshape(2 * H)
    ten = jnp.pad(W_tenant, ((0, TEN_PAD - W_tenant.shape[0]), (0, 0)))
    return _run(ids, cidx, pos, typ, ten, W_word)


# speedup vs baseline: 2.5715x; 1.3153x over previous
import jax
import jax.numpy as jnp
from jax import lax
from jax.experimental import pallas as pl
from jax.experimental.pallas import tpu as pltpu
from jax.experimental.pallas import tpu_sc as plsc

B = 1024
S = 200
H = 128
SPLIT_A = 104
SPLIT_B = S - SPLIT_A
NC = 2
NS = 16
NW = NC * NS
ROWS_PER_W = B // NW
LANES = 16
KCH = H // LANES
NQ = S // LANES
TAIL = S - NQ * LANES
TEN_PAD = 104


def _body(ids_h, cidx_h, pos_h, typ_h, ten_h, word_h, out_h,
          pos_v, combo_v, typ_v, ten_v, acc_v, idx_a, idx_b, cidx_v, gsem):
    c = lax.axis_index("c")
    s = lax.axis_index("s")
    wid = s * NC + c

    pltpu.sync_copy(pos_h, pos_v)
    pltpu.sync_copy(typ_h, typ_v)
    pltpu.sync_copy(ten_h, ten_v)

    def build(t, carry):
        for half in range(2):
            for k in range(KCH):
                sl = pl.ds(k * LANES, LANES)
                combo_v[half * 100 + t, sl] = (
                    ten_v[t, sl] + typ_v[pl.ds(half * H + k * LANES, LANES)])
        return carry
    lax.fori_loop(0, 100, build, 0)

    def do_token(t, ct):
        for k in range(KCH):
            sl = pl.ds(k * LANES, LANES)
            acc_v[t, sl] = acc_v[t, sl] + pos_v[t, sl] + combo_v[ct, sl]

    def row(r, carry):
        b = wid * ROWS_PER_W + r
        base = b * S
        with jax.named_scope("gid"):
            pltpu.sync_copy(ids_h.at[pl.ds(base, SPLIT_A)], idx_a)
            pltpu.sync_copy(ids_h.at[pl.ds(base + SPLIT_A, SPLIT_B)], idx_b)
            pltpu.sync_copy(cidx_h.at[pl.ds(base, S)], cidx_v.at[pl.ds(0, S)])
        with jax.named_scope("gath"):
            ga = pltpu.async_copy(word_h.at[idx_a],
                                  acc_v.at[pl.ds(0, SPLIT_A)], gsem)
            gb = pltpu.async_copy(word_h.at[idx_b],
                                  acc_v.at[pl.ds(SPLIT_A, SPLIT_B)], gsem)
            ga.wait()
            gb.wait()

        with jax.named_scope("comp"):
            @plsc.parallel_loop(0, NQ)
            def group(q):
                t0 = q * LANES
                chunk = cidx_v[pl.ds(t0, LANES)]
                for i in range(LANES):
                    do_token(t0 + i, chunk[i])

            tail_chunk = cidx_v[pl.ds(NQ * LANES, LANES)]
            for i in range(TAIL):
                do_token(NQ * LANES + i, tail_chunk[i])

        with jax.named_scope("wb"):
            pltpu.sync_copy(acc_v, out_h.at[b])
        return carry
    lax.fori_loop(0, ROWS_PER_W, row, 0)


@jax.jit
def _run(ids, cidx, pos, typ, ten, word):
    mesh = plsc.VectorSubcoreMesh(core_axis_name="c", subcore_axis_name="s")
    return pl.kernel(
        _body,
        out_type=jax.ShapeDtypeStruct((B, S, H), jnp.float32),
        mesh=mesh,
        scratch_types=[
            pltpu.VMEM((S, H), jnp.float32),
            pltpu.VMEM((S, H), jnp.float32),
            pltpu.VMEM((2 * H,), jnp.float32),
            pltpu.VMEM((TEN_PAD, H), jnp.float32),
            pltpu.VMEM((S, H), jnp.float32),
            pltpu.VMEM((SPLIT_A,), jnp.int32),
            pltpu.VMEM((SPLIT_B,), jnp.int32),
            pltpu.VMEM(((NQ + 1) * LANES,), jnp.int32),
            pltpu.SemaphoreType.DMA,
        ],
    )(ids, cidx, pos, typ, ten, word)


def kernel(input_ids, token_type_ids, tenant_ids, W_word, W_pos, W_type, W_tenant):
    ids = input_ids.astype(jnp.int32).reshape(B * S)
    cidx = (token_type_ids.astype(jnp.int32) * 100
            + tenant_ids.astype(jnp.int32)).reshape(B * S)
    pos = W_pos[:S]
    typ = W_type.reshape(2 * H)
    ten = jnp.pad(W_tenant, ((0, TEN_PAD - W_tenant.shape[0]), (0, 0)))
    return _run(ids, cidx, pos, typ, ten, W_word)
